# baseline (speedup 1.0000x reference)
import jax
import jax.numpy as jnp
from jax.experimental import pallas as pl


def _probe_copy(x):
    def body(x_ref, o_ref):
        o_ref[...] = x_ref[...]
    return pl.pallas_call(
        body, out_shape=jax.ShapeDtypeStruct(x.shape, x.dtype))(x)


def _bilin(flow_maps, x, y, bidx):
    Bq, Dq, Hq, Wq, _ = flow_maps.shape
    x0 = jnp.clip(jnp.floor(x), 0, Wq - 2).astype(jnp.int32)
    y0 = jnp.clip(jnp.floor(y), 0, Hq - 2).astype(jnp.int32)
    dx = jnp.clip(x - x0, 0.0, 1.0)
    dy = jnp.clip(y - y0, 0.0, 1.0)
    b = jnp.broadcast_to(jnp.arange(Bq)[:, None], x.shape)
    f00 = flow_maps[b, bidx, y0, x0]
    f01 = flow_maps[b, bidx, y0, x0 + 1]
    f10 = flow_maps[b, bidx, y0 + 1, x0]
    f11 = flow_maps[b, bidx, y0 + 1, x0 + 1]
    w00 = ((1 - dx) * (1 - dy))[..., None]
    w01 = (dx * (1 - dy))[..., None]
    w10 = ((1 - dx) * dy)[..., None]
    w11 = (dx * dy)[..., None]
    return f00 * w00 + f01 * w01 + f10 * w10 + f11 * w11


def _iwe(xw, yw, ts, pol, Hq, Wq):
    Bq, Nq, R = xw.shape
    size = Bq * 2 * R * Hq * Wq
    b = jnp.broadcast_to(jnp.arange(Bq)[:, None, None], xw.shape)
    ref_ = jnp.broadcast_to(jnp.arange(R)[None, None, :], xw.shape)
    polb = jnp.broadcast_to(pol[..., None], xw.shape)
    base_idx = ((b * 2 + polb) * R + ref_) * (Hq * Wq)
    xf = jnp.floor(xw)
    yf = jnp.floor(yw)
    ax = xw - xf
    ay = yw - yf
    x0 = xf.astype(jnp.int32)
    y0 = yf.astype(jnp.int32)
    inb = (xw >= 0) & (xw <= Wq - 1) & (yw >= 0) & (yw <= Hq - 1)
    iwe = jnp.zeros((size,), dtype=xw.dtype)
    iwt = jnp.zeros((size,), dtype=xw.dtype)
    corners = [
        (x0, y0, (1 - ax) * (1 - ay)),
        (x0 + 1, y0, ax * (1 - ay)),
        (x0, y0 + 1, (1 - ax) * ay),
        (x0 + 1, y0 + 1, ax * ay),
    ]
    for xc, yc, wgt in corners:
        valid = inb & (xc >= 0) & (xc <= Wq - 1) & (yc >= 0) & (yc <= Hq - 1)
        xcc = jnp.clip(xc, 0, Wq - 1)
        ycc = jnp.clip(yc, 0, Hq - 1)
        wv = wgt * valid.astype(xw.dtype)
        idx = (base_idx + ycc * Wq + xcc).ravel()
        iwe = iwe.at[idx].add(wv.ravel())
        iwt = iwt.at[idx].add((wv * ts).ravel())
    return iwe.reshape(Bq, 2, R, Hq, Wq), iwt.reshape(Bq, 2, R, Hq, Wq)


def kernel(events, flow_maps):
    Bq, Dq, Hq, Wq, _ = flow_maps.shape
    base = float(Dq)
    x = events[..., 0]
    y = events[..., 1]
    t = events[..., 2]
    p = events[..., 4]
    bidx = jnp.clip(jnp.floor(t), 0, Dq - 1).astype(jnp.int32)
    flow = _bilin(flow_maps, x, y, bidx)
    u = flow[..., 0]
    v = flow[..., 1]
    trefs = jnp.arange(Dq + 1, dtype=events.dtype)
    dt = trefs[None, None, :] - t[..., None]
    xw = x[..., None] + dt * u[..., None]
    yw = y[..., None] + dt * v[..., None]
    ts = jnp.abs(dt) / base
    pol = (p > 0.5).astype(jnp.int32)
    iwe, iwt = _iwe(xw, yw, ts, pol, Hq, Wq)
    iwe_neg, iwe_pos = iwe[:, 0], iwe[:, 1]
    iwt_neg, iwt_pos = iwt[:, 0], iwt[:, 1]
    iwat_neg = iwt_neg / (iwe_neg + 1e-9)
    iwat_pos = iwt_pos / (iwe_pos + 1e-9)
    inside = ((iwe_neg + iwe_pos) > 0).astype(events.dtype).reshape(Bq, Dq + 1, -1).sum(2) + 1e-9
    loss = (iwat_neg ** 2).reshape(Bq, Dq + 1, -1).sum(2) + (iwat_pos ** 2).reshape(Bq, Dq + 1, -1).sum(2)
    loss = loss / inside
    return _probe_copy(loss)


# speedup vs baseline: 13.6296x; 13.6296x over previous
import functools

import jax
import jax.numpy as jnp
from jax import lax
from jax.experimental import pallas as pl
from jax.experimental.pallas import tpu as pltpu
from jax.experimental.pallas import tpu_sc as plsc

NC = 2
NS = 16
NW = NC * NS
LANES = 16

CH_A = 128
CH_B = 2000


def _iota16():
    return lax.iota(jnp.int32, 16)


def _flow_sample_kernel(BN, Dq, Hq, Wq):
    HW = Hq * Wq
    DHW = Dq * HW
    n_chunks = BN // CH_A
    n_rounds = (n_chunks + NW - 1) // NW
    mesh = plsc.VectorSubcoreMesh(core_axis_name="c", subcore_axis_name="s")

    idx_t = [pltpu.VMEM((CH_A,), jnp.int32)] * 8
    fbuf_t = [pltpu.VMEM((CH_A,), jnp.float32)] * 8
    wbuf_t = [pltpu.VMEM((CH_A,), jnp.float32)] * 4

    @functools.partial(
        pl.kernel,
        mesh=mesh,
        out_type=(jax.ShapeDtypeStruct((BN,), jnp.float32),
                  jax.ShapeDtypeStruct((BN,), jnp.float32)),
        compiler_params=pltpu.CompilerParams(needs_layout_passes=False),
        scratch_types=[pltpu.VMEM((CH_A * 5,), jnp.float32)] + idx_t + fbuf_t
        + wbuf_t + [
            pltpu.VMEM((CH_A,), jnp.float32),
            pltpu.VMEM((CH_A,), jnp.float32),
            pltpu.SemaphoreType.DMA,
        ],
    )
    def flow_kernel(ev_hbm, flow_hbm, u_hbm, v_hbm,
                    evb, iu00, iv00, iu01, iv01, iu10, iv10, iu11, iv11,
                    fu00, fv00, fu01, fv01, fu10, fv10, fu11, fv11,
                    w00b, w01b, w10b, w11b, uo, vo, sem):
        wid = lax.axis_index("s") * NC + lax.axis_index("c")
        iota = _iota16()
        n_grp = CH_A // LANES

        def round_body(j, _):
            chunk = wid + NW * j

            @pl.when(chunk < n_chunks)
            def _():
                base = chunk * CH_A
                pltpu.sync_copy(ev_hbm.at[pl.ds(base * 5, CH_A * 5)], evb)

                def grp1(g, _):
                    s = g * LANES
                    rows5 = (iota + s) * 5
                    x = plsc.load_gather(evb, [rows5])
                    y = plsc.load_gather(evb, [rows5 + 1])
                    t = plsc.load_gather(evb, [rows5 + 2])
                    x0 = jnp.clip(x.astype(jnp.int32), 0, Wq - 2)
                    y0 = jnp.clip(y.astype(jnp.int32), 0, Hq - 2)
                    dx = jnp.clip(x - x0.astype(jnp.float32), 0.0, 1.0)
                    dy = jnp.clip(y - y0.astype(jnp.float32), 0.0, 1.0)
                    bidx = jnp.clip(t.astype(jnp.int32), 0, Dq - 1)
                    bvec = (base + iota + s) // (BN // 4)
                    u00 = 2 * (bvec * DHW + bidx * HW + y0 * Wq + x0)
                    iu00[pl.ds(s, LANES)] = u00
                    iv00[pl.ds(s, LANES)] = u00 + 1
                    iu01[pl.ds(s, LANES)] = u00 + 2
                    iv01[pl.ds(s, LANES)] = u00 + 3
                    iu10[pl.ds(s, LANES)] = u00 + 2 * Wq
                    iv10[pl.ds(s, LANES)] = u00 + 2 * Wq + 1
                    iu11[pl.ds(s, LANES)] = u00 + 2 * Wq + 2
                    iv11[pl.ds(s, LANES)] = u00 + 2 * Wq + 3
                    omdx = 1.0 - dx
                    omdy = 1.0 - dy
                    w00b[pl.ds(s, LANES)] = omdx * omdy
                    w01b[pl.ds(s, LANES)] = dx * omdy
                    w10b[pl.ds(s, LANES)] = omdx * dy
                    w11b[pl.ds(s, LANES)] = dx * dy
                    return _

                lax.fori_loop(0, n_grp, grp1, None)

                handles = [
                    pltpu.async_copy(flow_hbm.at[ib], fb, sem)
                    for ib, fb in ((iu00, fu00), (iv00, fv00),
                                   (iu01, fu01), (iv01, fv01),
                                   (iu10, fu10), (iv10, fv10),
                                   (iu11, fu11), (iv11, fv11))
                ]
                for h in handles:
                    h.wait()

                def grp2(g, _):
                    s = g * LANES
                    sl = pl.ds(s, LANES)
                    w00 = w00b[sl]
                    w01 = w01b[sl]
                    w10 = w10b[sl]
                    w11 = w11b[sl]
                    uo[sl] = (w00 * fu00[sl] + w01 * fu01[sl]
                              + w10 * fu10[sl] + w11 * fu11[sl])
                    vo[sl] = (w00 * fv00[sl] + w01 * fv01[sl]
                              + w10 * fv10[sl] + w11 * fv11[sl])
                    return _

                lax.fori_loop(0, n_grp, grp2, None)

                pltpu.sync_copy(uo, u_hbm.at[pl.ds(base, CH_A)])
                pltpu.sync_copy(vo, v_hbm.at[pl.ds(base, CH_A)])

            return _

        lax.fori_loop(0, n_rounds, round_body, None)

    return flow_kernel


def _hist_kernel(Bq, Nq, Dq, Hq, Wq):
    R = Dq + 1
    HHALF = Hq // 2
    PLANE = HHALF * Wq
    n_tasks = Bq * 2 * R * 2
    per_w = (n_tasks + NW - 1) // NW
    n_chunks = Nq // CH_B
    inv_base = 1.0 / float(Dq)
    mesh = plsc.VectorSubcoreMesh(core_axis_name="c", subcore_axis_name="s")
    out_sds = jax.ShapeDtypeStruct((Bq, 2, R, 2, PLANE), jnp.float32)

    @functools.partial(
        pl.kernel,
        mesh=mesh,
        out_type=(out_sds, out_sds),
        compiler_params=pltpu.CompilerParams(needs_layout_passes=False),
        scratch_types=[
            pltpu.VMEM((CH_B * 5,), jnp.float32),
            pltpu.VMEM((CH_B,), jnp.float32),
            pltpu.VMEM((CH_B,), jnp.float32),
            pltpu.VMEM((PLANE,), jnp.float32),
            pltpu.VMEM((PLANE,), jnp.float32),
        ],
    )
    def hist_kernel(ev_hbm, u_hbm, v_hbm, iwe_hbm, iwt_hbm, evb, ub, vb, eacc, tacc):
        wid = lax.axis_index("s") * NC + lax.axis_index("c")
        iota = _iota16()
        zero16 = jnp.zeros((16,), jnp.float32)
        n_grp = CH_B // LANES

        def task_body(k, _):
            task = wid + NW * k

            @pl.when(task < n_tasks)
            def _():
                b = task // (2 * R * 2)
                rem = task % (2 * R * 2)
                pol = rem // (R * 2)
                rem2 = rem % (R * 2)
                r = rem2 // 2
                half = rem2 % 2
                rf = r.astype(jnp.float32)
                pol_pos = pol > 0
                ybase = half * HHALF

                def zero_body(i, _):
                    eacc[pl.ds(i * LANES, LANES)] = zero16
                    tacc[pl.ds(i * LANES, LANES)] = zero16
                    return _

                lax.fori_loop(0, PLANE // LANES, zero_body, None)

                def chunk_body(ci, _):
                    off = b * Nq + ci * CH_B
                    pltpu.sync_copy(ev_hbm.at[pl.ds(off * 5, CH_B * 5)], evb)
                    pltpu.sync_copy(u_hbm.at[pl.ds(off, CH_B)], ub)
                    pltpu.sync_copy(v_hbm.at[pl.ds(off, CH_B)], vb)

                    def grp(g, _):
                        s = g * LANES
                        rows5 = (iota + s) * 5
                        x = plsc.load_gather(evb, [rows5])
                        y = plsc.load_gather(evb, [rows5 + 1])
                        t = plsc.load_gather(evb, [rows5 + 2])
                        p = plsc.load_gather(evb, [rows5 + 4])
                        u = ub[pl.ds(s, LANES)]
                        v = vb[pl.ds(s, LANES)]
                        dt = rf - t
                        xw = x + dt * u
                        yw = y + dt * v
                        ts = jnp.abs(dt) * inv_base
                        inb = ((xw >= 0.0) & (xw <= float(Wq - 1))
                               & (yw >= 0.0) & (yw <= float(Hq - 1)))
                        pm = inb & ((p > 0.5) == pol_pos)
                        xs = jnp.clip(xw, 0.0, float(Wq - 1))
                        ys = jnp.clip(yw, 0.0, float(Hq - 1))
                        x0 = xs.astype(jnp.int32)
                        y0 = ys.astype(jnp.int32)
                        ax = jnp.clip(xs - x0.astype(jnp.float32), 0.0, 1.0)
                        ay = jnp.clip(ys - y0.astype(jnp.float32), 0.0, 1.0)
                        wx = (1.0 - ax, ax)
                        wy = (1.0 - ay, ay)
                        for dxc in (0, 1):
                            for dyc in (0, 1):
                                cx = x0 + dxc
                                cy = y0 + dyc
                                cm = pm
                                if dxc:
                                    cm = cm & (cx <= Wq - 1)
                                if dyc:
                                    cm = cm & (cy <= Hq - 1)
                                ly = cy - ybase
                                cm = cm & (ly >= 0) & (ly <= HHALF - 1)
                                lyc = jnp.clip(ly, 0, HHALF - 1)
                                cxc = jnp.minimum(cx, Wq - 1)
                                lidx = lyc * Wq + cxc
                                wv = wx[dxc] * wy[dyc]
                                plsc.addupdate_scatter(eacc, [lidx], wv, mask=cm)
                                plsc.addupdate_scatter(tacc, [lidx], wv * ts,
                                                       mask=cm)
                        return _

                    lax.fori_loop(0, n_grp, grp, None)
                    return _

                lax.fori_loop(0, n_chunks, chunk_body, None)

                pltpu.sync_copy(eacc, iwe_hbm.at[b, pol, r, half])
                pltpu.sync_copy(tacc, iwt_hbm.at[b, pol, r, half])

            return _

        lax.fori_loop(0, per_w, task_body, None)

    return hist_kernel


def _loss_body(iwe_ref, iwt_ref, loss_ref):
    e = iwe_ref[0, :, 0]
    t = iwt_ref[0, :, 0]
    a = t / (e + 1e-9)
    s = jnp.sum(a * a)
    inside = jnp.sum(((e[0] + e[1]) > 0.0).astype(jnp.float32)) + 1e-9
    loss_ref[0, 0] = jnp.full((8, 128), s / inside, jnp.float32)


def _loss_kernel(Bq, R, Hq, Wq):
    spec = pl.BlockSpec((1, 2, 1, Hq, Wq), lambda i, j: (i, 0, j, 0, 0))
    return pl.pallas_call(
        _loss_body,
        grid=(Bq, R),
        in_specs=[spec, spec],
        out_specs=pl.BlockSpec((1, 1, 8, 128), lambda i, j: (i, j, 0, 0)),
        out_shape=jax.ShapeDtypeStruct((Bq, R, 8, 128), jnp.float32),
    )


def kernel(events, flow_maps):
    Bq, Dq, Hq, Wq, _ = flow_maps.shape
    _, Nq, _ = events.shape
    R = Dq + 1
    BN = Bq * Nq
    ev = events.reshape(BN * 5)
    flow2 = flow_maps.reshape(Bq * Dq * Hq * Wq * 2)
    uu, vv = _flow_sample_kernel(BN, Dq, Hq, Wq)(ev, flow2)
    iwe4, iwt4 = _hist_kernel(Bq, Nq, Dq, Hq, Wq)(ev, uu, vv)
    iwe = iwe4.reshape(Bq, 2, R, Hq, Wq)
    iwt = iwt4.reshape(Bq, 2, R, Hq, Wq)
    return _loss_kernel(Bq, R, Hq, Wq)(iwe, iwt)[:, :, 0, 0]


# speedup vs baseline: 13.9561x; 1.0240x over previous
import functools

import jax
import jax.numpy as jnp
from jax import lax
from jax.experimental import pallas as pl
from jax.experimental.pallas import tpu as pltpu
from jax.experimental.pallas import tpu_sc as plsc

NC = 2
NS = 16
NW = NC * NS
LANES = 16

CH_A = 128
CH_B = 2000


def _iota16():
    return lax.iota(jnp.int32, 16)


def _flow_sample_kernel(BN, Dq, Hq, Wq):
    HW = Hq * Wq
    DHW = Dq * HW
    n_chunks = BN // CH_A
    n_rounds = (n_chunks + NW - 1) // NW
    mesh = plsc.VectorSubcoreMesh(core_axis_name="c", subcore_axis_name="s")

    idx_t = [pltpu.VMEM((CH_A,), jnp.int32)] * 8
    fbuf_t = [pltpu.VMEM((CH_A,), jnp.float32)] * 8
    wbuf_t = [pltpu.VMEM((CH_A,), jnp.float32)] * 4

    @functools.partial(
        pl.kernel,
        mesh=mesh,
        out_type=(jax.ShapeDtypeStruct((BN,), jnp.float32),
                  jax.ShapeDtypeStruct((BN,), jnp.float32)),
        compiler_params=pltpu.CompilerParams(needs_layout_passes=False),
        scratch_types=[pltpu.VMEM((CH_A * 5,), jnp.float32)] + idx_t + fbuf_t
        + wbuf_t + [
            pltpu.VMEM((CH_A,), jnp.float32),
            pltpu.VMEM((CH_A,), jnp.float32),
            pltpu.SemaphoreType.DMA,
        ],
    )
    def flow_kernel(ev_hbm, flow_hbm, u_hbm, v_hbm,
                    evb, iu00, iv00, iu01, iv01, iu10, iv10, iu11, iv11,
                    fu00, fv00, fu01, fv01, fu10, fv10, fu11, fv11,
                    w00b, w01b, w10b, w11b, uo, vo, sem):
        wid = lax.axis_index("s") * NC + lax.axis_index("c")
        iota = _iota16()
        n_grp = CH_A // LANES

        def round_body(j, _):
            chunk = wid + NW * j

            @pl.when(chunk < n_chunks)
            def _():
                base = chunk * CH_A
                pltpu.sync_copy(ev_hbm.at[pl.ds(base * 5, CH_A * 5)], evb)

                def grp1(g, _):
                    s = g * LANES
                    rows5 = (iota + s) * 5
                    x = plsc.load_gather(evb, [rows5])
                    y = plsc.load_gather(evb, [rows5 + 1])
                    t = plsc.load_gather(evb, [rows5 + 2])
                    x0 = jnp.clip(x.astype(jnp.int32), 0, Wq - 2)
                    y0 = jnp.clip(y.astype(jnp.int32), 0, Hq - 2)
                    dx = jnp.clip(x - x0.astype(jnp.float32), 0.0, 1.0)
                    dy = jnp.clip(y - y0.astype(jnp.float32), 0.0, 1.0)
                    bidx = jnp.clip(t.astype(jnp.int32), 0, Dq - 1)
                    bvec = (base + iota + s) // (BN // 4)
                    u00 = 2 * (bvec * DHW + bidx * HW + y0 * Wq + x0)
                    iu00[pl.ds(s, LANES)] = u00
                    iv00[pl.ds(s, LANES)] = u00 + 1
                    iu01[pl.ds(s, LANES)] = u00 + 2
                    iv01[pl.ds(s, LANES)] = u00 + 3
                    iu10[pl.ds(s, LANES)] = u00 + 2 * Wq
                    iv10[pl.ds(s, LANES)] = u00 + 2 * Wq + 1
                    iu11[pl.ds(s, LANES)] = u00 + 2 * Wq + 2
                    iv11[pl.ds(s, LANES)] = u00 + 2 * Wq + 3
                    omdx = 1.0 - dx
                    omdy = 1.0 - dy
                    w00b[pl.ds(s, LANES)] = omdx * omdy
                    w01b[pl.ds(s, LANES)] = dx * omdy
                    w10b[pl.ds(s, LANES)] = omdx * dy
                    w11b[pl.ds(s, LANES)] = dx * dy
                    return _

                lax.fori_loop(0, n_grp, grp1, None)

                handles = [
                    pltpu.async_copy(flow_hbm.at[ib], fb, sem)
                    for ib, fb in ((iu00, fu00), (iv00, fv00),
                                   (iu01, fu01), (iv01, fv01),
                                   (iu10, fu10), (iv10, fv10),
                                   (iu11, fu11), (iv11, fv11))
                ]
                for h in handles:
                    h.wait()

                def grp2(g, _):
                    s = g * LANES
                    sl = pl.ds(s, LANES)
                    w00 = w00b[sl]
                    w01 = w01b[sl]
                    w10 = w10b[sl]
                    w11 = w11b[sl]
                    uo[sl] = (w00 * fu00[sl] + w01 * fu01[sl]
                              + w10 * fu10[sl] + w11 * fu11[sl])
                    vo[sl] = (w00 * fv00[sl] + w01 * fv01[sl]
                              + w10 * fv10[sl] + w11 * fv11[sl])
                    return _

                lax.fori_loop(0, n_grp, grp2, None)

                pltpu.sync_copy(uo, u_hbm.at[pl.ds(base, CH_A)])
                pltpu.sync_copy(vo, v_hbm.at[pl.ds(base, CH_A)])

            return _

        lax.fori_loop(0, n_rounds, round_body, None)

    return flow_kernel


def _hist_kernel(Bq, Nq, Dq, Hq, Wq):
    R = Dq + 1
    QROWS = Hq // 4
    PLANE = QROWS * Wq
    ACC = 2 * PLANE
    n_tasks = Bq * R * 4
    per_w = (n_tasks + NW - 1) // NW
    n_chunks = Nq // CH_B
    inv_base = 1.0 / float(Dq)
    mesh = plsc.VectorSubcoreMesh(core_axis_name="c", subcore_axis_name="s")

    @functools.partial(
        pl.kernel,
        mesh=mesh,
        out_type=jax.ShapeDtypeStruct((n_tasks * 32,), jnp.float32),
        compiler_params=pltpu.CompilerParams(needs_layout_passes=False),
        scratch_types=[
            pltpu.VMEM((CH_B * 5,), jnp.float32),
            pltpu.VMEM((CH_B,), jnp.float32),
            pltpu.VMEM((CH_B,), jnp.float32),
            pltpu.VMEM((ACC,), jnp.float32),
            pltpu.VMEM((ACC,), jnp.float32),
            pltpu.VMEM((16,), jnp.float32),
            pltpu.VMEM((16,), jnp.float32),
        ],
    )
    def hist_kernel(ev_hbm, u_hbm, v_hbm, part_hbm,
                    evb, ub, vb, eacc, tacc, sb, cb):
        wid = lax.axis_index("s") * NC + lax.axis_index("c")
        iota = _iota16()
        zero16 = jnp.zeros((16,), jnp.float32)
        n_grp = CH_B // LANES

        def task_body(k, _):
            task = wid + NW * k

            @pl.when(task < n_tasks)
            def _():
                b = task // (R * 4)
                rem = task % (R * 4)
                r = rem // 4
                q = rem % 4
                rf = r.astype(jnp.float32)
                ybase = q * QROWS

                def zero_body(i, _):
                    eacc[pl.ds(i * LANES, LANES)] = zero16
                    tacc[pl.ds(i * LANES, LANES)] = zero16
                    return _

                lax.fori_loop(0, ACC // LANES, zero_body, None)

                def chunk_body(ci, _):
                    off = b * Nq + ci * CH_B
                    pltpu.sync_copy(ev_hbm.at[pl.ds(off * 5, CH_B * 5)], evb)
                    pltpu.sync_copy(u_hbm.at[pl.ds(off, CH_B)], ub)
                    pltpu.sync_copy(v_hbm.at[pl.ds(off, CH_B)], vb)

                    def grp(g, _):
                        s = g * LANES
                        rows5 = (iota + s) * 5
                        x = plsc.load_gather(evb, [rows5])
                        y = plsc.load_gather(evb, [rows5 + 1])
                        t = plsc.load_gather(evb, [rows5 + 2])
                        p = plsc.load_gather(evb, [rows5 + 4])
                        u = ub[pl.ds(s, LANES)]
                        v = vb[pl.ds(s, LANES)]
                        dt = rf - t
                        xw = x + dt * u
                        yw = y + dt * v
                        ts = jnp.abs(dt) * inv_base
                        inb = ((xw >= 0.0) & (xw <= float(Wq - 1))
                               & (yw >= 0.0) & (yw <= float(Hq - 1)))
                        pbase = jnp.where(p > 0.5, PLANE, 0)
                        xs = jnp.clip(xw, 0.0, float(Wq - 1))
                        ys = jnp.clip(yw, 0.0, float(Hq - 1))
                        x0 = xs.astype(jnp.int32)
                        y0 = ys.astype(jnp.int32)
                        ax = jnp.clip(xs - x0.astype(jnp.float32), 0.0, 1.0)
                        ay = jnp.clip(ys - y0.astype(jnp.float32), 0.0, 1.0)
                        wx = (1.0 - ax, ax)
                        wy = (1.0 - ay, ay)
                        for dxc in (0, 1):
                            for dyc in (0, 1):
                                cx = x0 + dxc
                                cy = y0 + dyc
                                cm = inb
                                if dxc:
                                    cm = cm & (cx <= Wq - 1)
                                if dyc:
                                    cm = cm & (cy <= Hq - 1)
                                ly = cy - ybase
                                cm = cm & (ly >= 0) & (ly <= QROWS - 1)
                                lyc = jnp.clip(ly, 0, QROWS - 1)
                                cxc = jnp.minimum(cx, Wq - 1)
                                lidx = pbase + lyc * Wq + cxc
                                wv = wx[dxc] * wy[dyc]
                                plsc.addupdate_scatter(eacc, [lidx], wv, mask=cm)
                                plsc.addupdate_scatter(tacc, [lidx], wv * ts,
                                                       mask=cm)
                        return _

                    lax.fori_loop(0, n_grp, grp, None)
                    return _

                lax.fori_loop(0, n_chunks, chunk_body, None)

                def red(i, carry):
                    ssum, cnt = carry
                    sl0 = pl.ds(i * LANES, LANES)
                    sl1 = pl.ds(PLANE + i * LANES, LANES)
                    e0 = eacc[sl0]
                    e1 = eacc[sl1]
                    a0 = tacc[sl0] / (e0 + 1e-9)
                    a1 = tacc[sl1] / (e1 + 1e-9)
                    ssum = ssum + a0 * a0 + a1 * a1
                    cnt = cnt + jnp.where((e0 + e1) > 0.0, 1.0, 0.0)
                    return (ssum, cnt)

                ssum, cnt = lax.fori_loop(0, PLANE // LANES, red,
                                          (zero16, zero16))
                sb[...] = ssum
                cb[...] = cnt
                pltpu.sync_copy(sb, part_hbm.at[pl.ds(task * 32, 16)])
                pltpu.sync_copy(cb, part_hbm.at[pl.ds(task * 32 + 16, 16)])

            return _

        lax.fori_loop(0, per_w, task_body, None)

    return hist_kernel


def _combine_body(part_ref, loss_ref):
    m = part_ref[...]
    col = lax.broadcasted_iota(jnp.int32, m.shape, 1)
    is_sum = (col % 32) < 16
    s = jnp.sum(jnp.where(is_sum, m, 0.0), axis=1)
    c = jnp.sum(jnp.where(is_sum, 0.0, m), axis=1)
    loss_ref[...] = jnp.broadcast_to((s / (c + 1e-9))[:, None], m.shape)


def _combine_kernel(n_br):
    return pl.pallas_call(
        _combine_body,
        out_shape=jax.ShapeDtypeStruct((n_br, 128), jnp.float32),
    )


def kernel(events, flow_maps):
    Bq, Dq, Hq, Wq, _ = flow_maps.shape
    _, Nq, _ = events.shape
    R = Dq + 1
    BN = Bq * Nq
    ev = events.reshape(BN * 5)
    flow2 = flow_maps.reshape(Bq * Dq * Hq * Wq * 2)
    uu, vv = _flow_sample_kernel(BN, Dq, Hq, Wq)(ev, flow2)
    part = _hist_kernel(Bq, Nq, Dq, Hq, Wq)(ev, uu, vv)
    loss = _combine_kernel(Bq * R)(part.reshape(Bq * R, 128))
    return loss[:, 0].reshape(Bq, R)


# speedup vs baseline: 34.8390x; 2.4963x over previous
import functools

import jax
import jax.numpy as jnp
from jax import lax
from jax.experimental import pallas as pl
from jax.experimental.pallas import tpu as pltpu
from jax.experimental.pallas import tpu_sc as plsc

NC = 2
NS = 16
NW = NC * NS
LANES = 16

CH_A = 128
CH_B = 2000


def _iota16():
    return lax.iota(jnp.int32, 16)


def _flow_sample_kernel(BN, Nq, Dq, Hq, Wq):
    n_chunks = BN // CH_A
    n_rounds = (n_chunks + NW - 1) // NW
    WSTRIDE = 512
    mesh = plsc.VectorSubcoreMesh(core_axis_name="c", subcore_axis_name="s")

    idx_t = [pltpu.VMEM((CH_A,), jnp.int32)] * 8
    fbuf_t = [pltpu.VMEM((CH_A,), jnp.float32)] * 8
    wbuf_t = [pltpu.VMEM((CH_A,), jnp.float32)] * 4

    @functools.partial(
        pl.kernel,
        mesh=mesh,
        out_type=(jax.ShapeDtypeStruct((BN,), jnp.float32),
                  jax.ShapeDtypeStruct((BN,), jnp.float32)),
        compiler_params=pltpu.CompilerParams(needs_layout_passes=False),
        scratch_types=[pltpu.VMEM((CH_A,), jnp.float32)] * 3 + idx_t + fbuf_t
        + wbuf_t + [
            pltpu.VMEM((CH_A,), jnp.float32),
            pltpu.VMEM((CH_A,), jnp.float32),
            pltpu.SemaphoreType.DMA,
        ],
    )
    def flow_kernel(x_hbm, y_hbm, t_hbm, flow_hbm, u_hbm, v_hbm,
                    xb, yb, tb,
                    iu00, iv00, iu01, iv01, iu10, iv10, iu11, iv11,
                    fu00, fv00, fu01, fv01, fu10, fv10, fu11, fv11,
                    w00b, w01b, w10b, w11b, uo, vo, sem):
        wid = lax.axis_index("s") * NC + lax.axis_index("c")
        iota = _iota16()
        n_grp = CH_A // LANES

        def round_body(j, _):
            chunk = wid + NW * j

            @pl.when(chunk < n_chunks)
            def _():
                base = chunk * CH_A
                pltpu.sync_copy(x_hbm.at[pl.ds(base, CH_A)], xb)
                pltpu.sync_copy(y_hbm.at[pl.ds(base, CH_A)], yb)
                pltpu.sync_copy(t_hbm.at[pl.ds(base, CH_A)], tb)

                def grp1(g, _):
                    s = g * LANES
                    sl = pl.ds(s, LANES)
                    x = xb[sl]
                    y = yb[sl]
                    t = tb[sl]
                    x0 = jnp.clip(x.astype(jnp.int32), 0, Wq - 2)
                    y0 = jnp.clip(y.astype(jnp.int32), 0, Hq - 2)
                    dx = jnp.clip(x - x0.astype(jnp.float32), 0.0, 1.0)
                    dy = jnp.clip(y - y0.astype(jnp.float32), 0.0, 1.0)
                    bidx = jnp.clip(t.astype(jnp.int32), 0, Dq - 1)
                    bvec = (base + iota + s) // Nq
                    col0 = (bvec * Dq + bidx) * (Wq * WSTRIDE) + x0 * WSTRIDE
                    y1 = y0 + 1
                    yt0 = ((y0 >> 7) << 8) + (y0 & 127)
                    yt1 = ((y1 >> 7) << 8) + (y1 & 127)
                    u00 = col0 + yt0
                    u10 = col0 + yt1
                    iu00[sl] = u00
                    iv00[sl] = u00 + 128
                    iu01[sl] = u00 + WSTRIDE
                    iv01[sl] = u00 + WSTRIDE + 128
                    iu10[sl] = u10
                    iv10[sl] = u10 + 128
                    iu11[sl] = u10 + WSTRIDE
                    iv11[sl] = u10 + WSTRIDE + 128
                    omdx = 1.0 - dx
                    omdy = 1.0 - dy
                    w00b[sl] = omdx * omdy
                    w01b[sl] = dx * omdy
                    w10b[sl] = omdx * dy
                    w11b[sl] = dx * dy
                    return _

                lax.fori_loop(0, n_grp, grp1, None)

                handles = [
                    pltpu.async_copy(flow_hbm.at[ib], fb, sem)
                    for ib, fb in ((iu00, fu00), (iv00, fv00),
                                   (iu01, fu01), (iv01, fv01),
                                   (iu10, fu10), (iv10, fv10),
                                   (iu11, fu11), (iv11, fv11))
                ]
                for h in handles:
                    h.wait()

                def grp2(g, _):
                    s = g * LANES
                    sl = pl.ds(s, LANES)
                    w00 = w00b[sl]
                    w01 = w01b[sl]
                    w10 = w10b[sl]
                    w11 = w11b[sl]
                    uo[sl] = (w00 * fu00[sl] + w01 * fu01[sl]
                              + w10 * fu10[sl] + w11 * fu11[sl])
                    vo[sl] = (w00 * fv00[sl] + w01 * fv01[sl]
                              + w10 * fv10[sl] + w11 * fv11[sl])
                    return _

                lax.fori_loop(0, n_grp, grp2, None)

                pltpu.sync_copy(uo, u_hbm.at[pl.ds(base, CH_A)])
                pltpu.sync_copy(vo, v_hbm.at[pl.ds(base, CH_A)])

            return _

        lax.fori_loop(0, n_rounds, round_body, None)

    return flow_kernel


def _hist_kernel(Bq, Nq, Dq, Hq, Wq):
    R = Dq + 1
    QROWS = Hq // 4
    PLANE = QROWS * Wq
    ACC = 2 * PLANE
    n_tasks = Bq * R * 4
    per_w = (n_tasks + NW - 1) // NW
    n_chunks = Nq // CH_B
    inv_base = 1.0 / float(Dq)
    mesh = plsc.VectorSubcoreMesh(core_axis_name="c", subcore_axis_name="s")

    @functools.partial(
        pl.kernel,
        mesh=mesh,
        out_type=jax.ShapeDtypeStruct((n_tasks * 32,), jnp.float32),
        compiler_params=pltpu.CompilerParams(needs_layout_passes=False),
        scratch_types=[pltpu.VMEM((CH_B,), jnp.float32)] * 6 + [
            pltpu.VMEM((ACC,), jnp.float32),
            pltpu.VMEM((ACC,), jnp.float32),
            pltpu.VMEM((16,), jnp.float32),
            pltpu.VMEM((16,), jnp.float32),
        ],
    )
    def hist_kernel(x_hbm, y_hbm, t_hbm, p_hbm, u_hbm, v_hbm, part_hbm,
                    xb, yb, tb, pb, ub, vb, eacc, tacc, sb, cb):
        wid = lax.axis_index("s") * NC + lax.axis_index("c")
        zero16 = jnp.zeros((16,), jnp.float32)
        n_grp = CH_B // LANES

        def task_body(k, _):
            task = wid + NW * k

            @pl.when(task < n_tasks)
            def _():
                b = task // (R * 4)
                rem = task % (R * 4)
                r = rem // 4
                q = rem % 4
                rf = r.astype(jnp.float32)
                ybase = q * QROWS

                def zero_body(i, _):
                    eacc[pl.ds(i * LANES, LANES)] = zero16
                    tacc[pl.ds(i * LANES, LANES)] = zero16
                    return _

                lax.fori_loop(0, ACC // LANES, zero_body, None)

                def chunk_body(ci, _):
                    off = b * Nq + ci * CH_B
                    osl = pl.ds(off, CH_B)
                    pltpu.sync_copy(x_hbm.at[osl], xb)
                    pltpu.sync_copy(y_hbm.at[osl], yb)
                    pltpu.sync_copy(t_hbm.at[osl], tb)
                    pltpu.sync_copy(p_hbm.at[osl], pb)
                    pltpu.sync_copy(u_hbm.at[osl], ub)
                    pltpu.sync_copy(v_hbm.at[osl], vb)

                    def grp(g, _):
                        s = g * LANES
                        sl = pl.ds(s, LANES)
                        x = xb[sl]
                        y = yb[sl]
                        t = tb[sl]
                        p = pb[sl]
                        u = ub[sl]
                        v = vb[sl]
                        dt = rf - t
                        xw = x + dt * u
                        yw = y + dt * v
                        ts = jnp.abs(dt) * inv_base
                        inb = ((xw >= 0.0) & (xw <= float(Wq - 1))
                               & (yw >= 0.0) & (yw <= float(Hq - 1)))
                        pbase = jnp.where(p > 0.5, PLANE, 0)
                        xs = jnp.clip(xw, 0.0, float(Wq - 1))
                        ys = jnp.clip(yw, 0.0, float(Hq - 1))
                        x0 = xs.astype(jnp.int32)
                        y0 = ys.astype(jnp.int32)
                        ax = jnp.clip(xs - x0.astype(jnp.float32), 0.0, 1.0)
                        ay = jnp.clip(ys - y0.astype(jnp.float32), 0.0, 1.0)
                        wx = (1.0 - ax, ax)
                        wy = (1.0 - ay, ay)
                        for dxc in (0, 1):
                            for dyc in (0, 1):
                                cx = x0 + dxc
                                cy = y0 + dyc
                                cm = inb
                                if dxc:
                                    cm = cm & (cx <= Wq - 1)
                                if dyc:
                                    cm = cm & (cy <= Hq - 1)
                                ly = cy - ybase
                                cm = cm & (ly >= 0) & (ly <= QROWS - 1)
                                lyc = jnp.clip(ly, 0, QROWS - 1)
                                cxc = jnp.minimum(cx, Wq - 1)
                                lidx = pbase + lyc * Wq + cxc
                                wv = wx[dxc] * wy[dyc]
                                plsc.addupdate_scatter(eacc, [lidx], wv, mask=cm)
                                plsc.addupdate_scatter(tacc, [lidx], wv * ts,
                                                       mask=cm)
                        return _

                    lax.fori_loop(0, n_grp, grp, None)
                    return _

                lax.fori_loop(0, n_chunks, chunk_body, None)

                def red(i, carry):
                    ssum, cnt = carry
                    sl0 = pl.ds(i * LANES, LANES)
                    sl1 = pl.ds(PLANE + i * LANES, LANES)
                    e0 = eacc[sl0]
                    e1 = eacc[sl1]
                    a0 = tacc[sl0] / (e0 + 1e-9)
                    a1 = tacc[sl1] / (e1 + 1e-9)
                    ssum = ssum + a0 * a0 + a1 * a1
                    cnt = cnt + jnp.where((e0 + e1) > 0.0, 1.0, 0.0)
                    return (ssum, cnt)

                ssum, cnt = lax.fori_loop(0, PLANE // LANES, red,
                                          (zero16, zero16))
                sb[...] = ssum
                cb[...] = cnt
                pltpu.sync_copy(sb, part_hbm.at[pl.ds(task * 32, 16)])
                pltpu.sync_copy(cb, part_hbm.at[pl.ds(task * 32 + 16, 16)])

            return _

        lax.fori_loop(0, per_w, task_body, None)

    return hist_kernel


def _combine_body(part_ref, loss_ref):
    m = part_ref[...]
    col = lax.broadcasted_iota(jnp.int32, m.shape, 1)
    is_sum = (col % 32) < 16
    s = jnp.sum(jnp.where(is_sum, m, 0.0), axis=1)
    c = jnp.sum(jnp.where(is_sum, 0.0, m), axis=1)
    loss_ref[...] = jnp.broadcast_to((s / (c + 1e-9))[:, None], m.shape)


def _combine_kernel(n_br):
    return pl.pallas_call(
        _combine_body,
        out_shape=jax.ShapeDtypeStruct((n_br, 128), jnp.float32),
    )


def kernel(events, flow_maps):
    Bq, Dq, Hq, Wq, _ = flow_maps.shape
    _, Nq, _ = events.shape
    R = Dq + 1
    BN = Bq * Nq
    xs = events[:, :, 0].reshape(BN)
    ys = events[:, :, 1].reshape(BN)
    tt = events[:, :, 2].reshape(BN)
    pp = events[:, :, 4].reshape(BN)
    fmp = (flow_maps.reshape(Bq, Dq, Hq // 128, 128, Wq, 2)
           .transpose(0, 1, 4, 2, 5, 3).reshape(-1))
    uu, vv = _flow_sample_kernel(BN, Nq, Dq, Hq, Wq)(xs, ys, tt, fmp)
    part = _hist_kernel(Bq, Nq, Dq, Hq, Wq)(xs, ys, tt, pp, uu, vv)
    loss = _combine_kernel(Bq * R)(part.reshape(Bq * R, 128))
    return loss[:, 0].reshape(Bq, R)


# speedup vs baseline: 63.5858x; 1.8251x over previous
import functools

import jax
import jax.numpy as jnp
from jax import lax
from jax.experimental import pallas as pl
from jax.experimental.pallas import tpu as pltpu
from jax.experimental.pallas import tpu_sc as plsc

NC = 2
NS = 16
NW = NC * NS
LANES = 16

CH_A = 128
CH_B = 2000


def _iota16():
    return lax.iota(jnp.int32, 16)


def _flow_sample_kernel(BN, Nq, Dq, Hq, Wq):
    n_chunks = BN // CH_A
    n_rounds = (n_chunks + NW - 1) // NW
    WSTRIDE = 512
    mesh = plsc.VectorSubcoreMesh(core_axis_name="c", subcore_axis_name="s")

    idx_t = [pltpu.VMEM((CH_A,), jnp.int32)] * 8
    fbuf_t = [pltpu.VMEM((CH_A,), jnp.float32)] * 8
    wbuf_t = [pltpu.VMEM((CH_A,), jnp.float32)] * 4

    @functools.partial(
        pl.kernel,
        mesh=mesh,
        out_type=(jax.ShapeDtypeStruct((BN,), jnp.float32),
                  jax.ShapeDtypeStruct((BN,), jnp.float32)),
        compiler_params=pltpu.CompilerParams(needs_layout_passes=False),
        scratch_types=[pltpu.VMEM((CH_A,), jnp.float32)] * 3 + idx_t + fbuf_t
        + wbuf_t + [
            pltpu.VMEM((CH_A,), jnp.float32),
            pltpu.VMEM((CH_A,), jnp.float32),
            pltpu.SemaphoreType.DMA,
        ],
    )
    def flow_kernel(x_hbm, y_hbm, t_hbm, flow_hbm, u_hbm, v_hbm,
                    xb, yb, tb,
                    iu00, iv00, iu01, iv01, iu10, iv10, iu11, iv11,
                    fu00, fv00, fu01, fv01, fu10, fv10, fu11, fv11,
                    w00b, w01b, w10b, w11b, uo, vo, sem):
        wid = lax.axis_index("s") * NC + lax.axis_index("c")
        iota = _iota16()
        n_grp = CH_A // LANES

        def round_body(j, _):
            chunk = wid + NW * j

            @pl.when(chunk < n_chunks)
            def _():
                base = chunk * CH_A
                pltpu.sync_copy(x_hbm.at[pl.ds(base, CH_A)], xb)
                pltpu.sync_copy(y_hbm.at[pl.ds(base, CH_A)], yb)
                pltpu.sync_copy(t_hbm.at[pl.ds(base, CH_A)], tb)

                def grp1(g, _):
                    s = g * LANES
                    sl = pl.ds(s, LANES)
                    x = xb[sl]
                    y = yb[sl]
                    t = tb[sl]
                    x0 = jnp.clip(x.astype(jnp.int32), 0, Wq - 2)
                    y0 = jnp.clip(y.astype(jnp.int32), 0, Hq - 2)
                    dx = jnp.clip(x - x0.astype(jnp.float32), 0.0, 1.0)
                    dy = jnp.clip(y - y0.astype(jnp.float32), 0.0, 1.0)
                    bidx = jnp.clip(t.astype(jnp.int32), 0, Dq - 1)
                    bvec = (base + iota + s) // Nq
                    col0 = (bvec * Dq + bidx) * (Wq * WSTRIDE) + x0 * WSTRIDE
                    y1 = y0 + 1
                    yt0 = ((y0 >> 7) << 8) + (y0 & 127)
                    yt1 = ((y1 >> 7) << 8) + (y1 & 127)
                    u00 = col0 + yt0
                    u10 = col0 + yt1
                    iu00[sl] = u00
                    iv00[sl] = u00 + 128
                    iu01[sl] = u00 + WSTRIDE
                    iv01[sl] = u00 + WSTRIDE + 128
                    iu10[sl] = u10
                    iv10[sl] = u10 + 128
                    iu11[sl] = u10 + WSTRIDE
                    iv11[sl] = u10 + WSTRIDE + 128
                    omdx = 1.0 - dx
                    omdy = 1.0 - dy
                    w00b[sl] = omdx * omdy
                    w01b[sl] = dx * omdy
                    w10b[sl] = omdx * dy
                    w11b[sl] = dx * dy
                    return _

                lax.fori_loop(0, n_grp, grp1, None)

                handles = [
                    pltpu.async_copy(flow_hbm.at[ib], fb, sem)
                    for ib, fb in ((iu00, fu00), (iv00, fv00),
                                   (iu01, fu01), (iv01, fv01),
                                   (iu10, fu10), (iv10, fv10),
                                   (iu11, fu11), (iv11, fv11))
                ]
                for h in handles:
                    h.wait()

                def grp2(g, _):
                    s = g * LANES
                    sl = pl.ds(s, LANES)
                    w00 = w00b[sl]
                    w01 = w01b[sl]
                    w10 = w10b[sl]
                    w11 = w11b[sl]
                    uo[sl] = (w00 * fu00[sl] + w01 * fu01[sl]
                              + w10 * fu10[sl] + w11 * fu11[sl])
                    vo[sl] = (w00 * fv00[sl] + w01 * fv01[sl]
                              + w10 * fv10[sl] + w11 * fv11[sl])
                    return _

                lax.fori_loop(0, n_grp, grp2, None)

                pltpu.sync_copy(uo, u_hbm.at[pl.ds(base, CH_A)])
                pltpu.sync_copy(vo, v_hbm.at[pl.ds(base, CH_A)])

            return _

        lax.fori_loop(0, n_rounds, round_body, None)

    return flow_kernel


def _hist_kernel(Bq, Nq, Dq, Hq, Wq):
    R = Dq + 1
    QROWS = Hq // 4
    PLANE = QROWS * Wq
    ACC = 2 * PLANE
    n_tasks = Bq * R * 4
    per_w = (n_tasks + NW - 1) // NW
    n_chunks = Nq // CH_B
    inv_base = 1.0 / float(Dq)
    mesh = plsc.VectorSubcoreMesh(core_axis_name="c", subcore_axis_name="s")

    @functools.partial(
        pl.kernel,
        mesh=mesh,
        out_type=jax.ShapeDtypeStruct((n_tasks * 32,), jnp.float32),
        compiler_params=pltpu.CompilerParams(needs_layout_passes=False),
        scratch_types=[pltpu.VMEM((CH_B,), jnp.float32)] * 12 + [
            pltpu.VMEM((ACC,), jnp.float32),
            pltpu.VMEM((ACC,), jnp.float32),
            pltpu.VMEM((16,), jnp.float32),
            pltpu.VMEM((16,), jnp.float32),
            pltpu.SemaphoreType.DMA,
            pltpu.SemaphoreType.DMA,
        ],
    )
    def hist_kernel(x_hbm, y_hbm, t_hbm, p_hbm, u_hbm, v_hbm, part_hbm,
                    xb0, yb0, tb0, pb0, ub0, vb0,
                    xb1, yb1, tb1, pb1, ub1, vb1,
                    eacc, tacc, sb, cb, sem0, sem1):
        wid = lax.axis_index("s") * NC + lax.axis_index("c")
        zero16 = jnp.zeros((16,), jnp.float32)
        n_grp = CH_B // LANES
        hbms = (x_hbm, y_hbm, t_hbm, p_hbm, u_hbm, v_hbm)
        bufsets = ((xb0, yb0, tb0, pb0, ub0, vb0),
                   (xb1, yb1, tb1, pb1, ub1, vb1))
        sems = (sem0, sem1)

        def fire(setn, b, ci):
            osl = pl.ds(b * Nq + ci * CH_B, CH_B)
            for h, buf in zip(hbms, bufsets[setn]):
                pltpu.async_copy(h.at[osl], buf, sems[setn])

        def drain(setn, b, ci):
            osl = pl.ds(b * Nq + ci * CH_B, CH_B)
            for h, buf in zip(hbms, bufsets[setn]):
                pltpu.make_async_copy(h.at[osl], buf, sems[setn]).wait()

        def task_body(k, _):
            task = wid + NW * k

            @pl.when(task < n_tasks)
            def _():
                b = task // (R * 4)
                rem = task % (R * 4)
                r = rem // 4
                q = rem % 4
                rf = r.astype(jnp.float32)
                ybase = q * QROWS

                fire(0, b, 0)

                def zero_body(i, _):
                    eacc[pl.ds(i * LANES, LANES)] = zero16
                    tacc[pl.ds(i * LANES, LANES)] = zero16
                    return _

                lax.fori_loop(0, ACC // LANES, zero_body, None)

                def compute(bufs):
                    xb, yb, tb, pb, ub, vb = bufs

                    def grp(g, _):
                        s = g * LANES
                        sl = pl.ds(s, LANES)
                        x = xb[sl]
                        y = yb[sl]
                        t = tb[sl]
                        p = pb[sl]
                        u = ub[sl]
                        v = vb[sl]
                        dt = rf - t
                        xw = x + dt * u
                        yw = y + dt * v
                        ts = jnp.abs(dt) * inv_base
                        inb = ((xw >= 0.0) & (xw <= float(Wq - 1))
                               & (yw >= 0.0) & (yw <= float(Hq - 1)))
                        pbase = jnp.where(p > 0.5, PLANE, 0)
                        xs = jnp.clip(xw, 0.0, float(Wq - 1))
                        ys = jnp.clip(yw, 0.0, float(Hq - 1))
                        x0 = xs.astype(jnp.int32)
                        y0 = ys.astype(jnp.int32)
                        ax = jnp.clip(xs - x0.astype(jnp.float32), 0.0, 1.0)
                        ay = jnp.clip(ys - y0.astype(jnp.float32), 0.0, 1.0)
                        wx = (1.0 - ax, ax)
                        wy = (1.0 - ay, ay)
                        for dxc in (0, 1):
                            for dyc in (0, 1):
                                cx = x0 + dxc
                                cy = y0 + dyc
                                cm = inb
                                if dxc:
                                    cm = cm & (cx <= Wq - 1)
                                if dyc:
                                    cm = cm & (cy <= Hq - 1)
                                ly = cy - ybase
                                cm = cm & (ly >= 0) & (ly <= QROWS - 1)
                                lyc = jnp.clip(ly, 0, QROWS - 1)
                                cxc = jnp.minimum(cx, Wq - 1)
                                lidx = pbase + lyc * Wq + cxc
                                wv = wx[dxc] * wy[dyc]
                                plsc.addupdate_scatter(eacc, [lidx], wv, mask=cm)
                                plsc.addupdate_scatter(tacc, [lidx], wv * ts,
                                                       mask=cm)
                        return _

                    lax.fori_loop(0, n_grp, grp, None)

                def pair_body(pair, _):
                    c0 = 2 * pair
                    drain(0, b, c0)
                    fire(1, b, c0 + 1)
                    compute(bufsets[0])
                    drain(1, b, c0 + 1)

                    @pl.when(c0 + 2 < n_chunks)
                    def _():
                        fire(0, b, c0 + 2)

                    compute(bufsets[1])
                    return _

                lax.fori_loop(0, n_chunks // 2, pair_body, None)

                def red(i, carry):
                    ssum, cnt = carry
                    sl0 = pl.ds(i * LANES, LANES)
                    sl1 = pl.ds(PLANE + i * LANES, LANES)
                    e0 = eacc[sl0]
                    e1 = eacc[sl1]
                    a0 = tacc[sl0] / (e0 + 1e-9)
                    a1 = tacc[sl1] / (e1 + 1e-9)
                    ssum = ssum + a0 * a0 + a1 * a1
                    cnt = cnt + jnp.where((e0 + e1) > 0.0, 1.0, 0.0)
                    return (ssum, cnt)

                ssum, cnt = lax.fori_loop(0, PLANE // LANES, red,
                                          (zero16, zero16))
                sb[...] = ssum
                cb[...] = cnt
                pltpu.sync_copy(sb, part_hbm.at[pl.ds(task * 32, 16)])
                pltpu.sync_copy(cb, part_hbm.at[pl.ds(task * 32 + 16, 16)])

            return _

        lax.fori_loop(0, per_w, task_body, None)

    return hist_kernel


def _combine_body(part_ref, loss_ref):
    m = part_ref[...]
    col = lax.broadcasted_iota(jnp.int32, m.shape, 1)
    is_sum = (col % 32) < 16
    s = jnp.sum(jnp.where(is_sum, m, 0.0), axis=1)
    c = jnp.sum(jnp.where(is_sum, 0.0, m), axis=1)
    loss_ref[...] = jnp.broadcast_to((s / (c + 1e-9))[:, None], m.shape)


def _combine_kernel(n_br):
    return pl.pallas_call(
        _combine_body,
        out_shape=jax.ShapeDtypeStruct((n_br, 128), jnp.float32),
    )


def kernel(events, flow_maps):
    Bq, Dq, Hq, Wq, _ = flow_maps.shape
    _, Nq, _ = events.shape
    R = Dq + 1
    BN = Bq * Nq
    xs = events[:, :, 0].reshape(BN)
    ys = events[:, :, 1].reshape(BN)
    tt = events[:, :, 2].reshape(BN)
    pp = events[:, :, 4].reshape(BN)
    fmp = (flow_maps.reshape(Bq, Dq, Hq // 128, 128, Wq, 2)
           .transpose(0, 1, 4, 2, 5, 3).reshape(-1))
    uu, vv = _flow_sample_kernel(BN, Nq, Dq, Hq, Wq)(xs, ys, tt, fmp)
    part = _hist_kernel(Bq, Nq, Dq, Hq, Wq)(xs, ys, tt, pp, uu, vv)
    loss = _combine_kernel(Bq * R)(part.reshape(Bq * R, 128))
    return loss[:, 0].reshape(Bq, R)


# speedup vs baseline: 78.3856x; 1.2328x over previous
import functools

import jax
import jax.numpy as jnp
from jax import lax
from jax.experimental import pallas as pl
from jax.experimental.pallas import tpu as pltpu
from jax.experimental.pallas import tpu_sc as plsc

NC = 2
NS = 16
NW = NC * NS
LANES = 16

CH_A = 640
GSUB = 128
CH_B = 2000


def _iota16():
    return lax.iota(jnp.int32, 16)


def _flow_sample_kernel(BN, Nq, Dq, Hq, Wq):
    n_chunks = BN // CH_A
    n_rounds = (n_chunks + NW - 1) // NW
    WSTRIDE = 512
    mesh = plsc.VectorSubcoreMesh(core_axis_name="c", subcore_axis_name="s")

    idx_t = [pltpu.VMEM((CH_A,), jnp.int32)] * 8
    fbuf_t = [pltpu.VMEM((CH_A,), jnp.float32)] * 8
    wbuf_t = [pltpu.VMEM((CH_A,), jnp.float32)] * 4

    @functools.partial(
        pl.kernel,
        mesh=mesh,
        out_type=(jax.ShapeDtypeStruct((BN,), jnp.float32),
                  jax.ShapeDtypeStruct((BN,), jnp.float32)),
        compiler_params=pltpu.CompilerParams(needs_layout_passes=False),
        scratch_types=[pltpu.VMEM((CH_A,), jnp.float32)] * 3 + idx_t + fbuf_t
        + wbuf_t + [
            pltpu.VMEM((CH_A,), jnp.float32),
            pltpu.VMEM((CH_A,), jnp.float32),
            pltpu.SemaphoreType.DMA,
        ],
    )
    def flow_kernel(x_hbm, y_hbm, t_hbm, flow_hbm, u_hbm, v_hbm,
                    xb, yb, tb,
                    iu00, iv00, iu01, iv01, iu10, iv10, iu11, iv11,
                    fu00, fv00, fu01, fv01, fu10, fv10, fu11, fv11,
                    w00b, w01b, w10b, w11b, uo, vo, sem):
        wid = lax.axis_index("s") * NC + lax.axis_index("c")
        iota = _iota16()
        n_grp = CH_A // LANES

        def round_body(j, _):
            chunk = wid + NW * j

            @pl.when(chunk < n_chunks)
            def _():
                base = chunk * CH_A
                pltpu.sync_copy(x_hbm.at[pl.ds(base, CH_A)], xb)
                pltpu.sync_copy(y_hbm.at[pl.ds(base, CH_A)], yb)
                pltpu.sync_copy(t_hbm.at[pl.ds(base, CH_A)], tb)

                def grp1(g, _):
                    s = g * LANES
                    sl = pl.ds(s, LANES)
                    x = xb[sl]
                    y = yb[sl]
                    t = tb[sl]
                    x0 = jnp.clip(x.astype(jnp.int32), 0, Wq - 2)
                    y0 = jnp.clip(y.astype(jnp.int32), 0, Hq - 2)
                    dx = jnp.clip(x - x0.astype(jnp.float32), 0.0, 1.0)
                    dy = jnp.clip(y - y0.astype(jnp.float32), 0.0, 1.0)
                    bidx = jnp.clip(t.astype(jnp.int32), 0, Dq - 1)
                    bvec = (base + iota + s) // Nq
                    col0 = (bvec * Dq + bidx) * (Wq * WSTRIDE) + x0 * WSTRIDE
                    y1 = y0 + 1
                    yt0 = ((y0 >> 7) << 8) + (y0 & 127)
                    yt1 = ((y1 >> 7) << 8) + (y1 & 127)
                    u00 = col0 + yt0
                    u10 = col0 + yt1
                    iu00[sl] = u00
                    iv00[sl] = u00 + 128
                    iu01[sl] = u00 + WSTRIDE
                    iv01[sl] = u00 + WSTRIDE + 128
                    iu10[sl] = u10
                    iv10[sl] = u10 + 128
                    iu11[sl] = u10 + WSTRIDE
                    iv11[sl] = u10 + WSTRIDE + 128
                    omdx = 1.0 - dx
                    omdy = 1.0 - dy
                    w00b[sl] = omdx * omdy
                    w01b[sl] = dx * omdy
                    w10b[sl] = omdx * dy
                    w11b[sl] = dx * dy
                    return _

                lax.fori_loop(0, n_grp, grp1, None)

                pairs = ((iu00, fu00), (iv00, fv00),
                         (iu01, fu01), (iv01, fv01),
                         (iu10, fu10), (iv10, fv10),
                         (iu11, fu11), (iv11, fv11))
                handles = []
                for ib, fb in pairs:
                    for j in range(CH_A // GSUB):
                        gsl = pl.ds(j * GSUB, GSUB)
                        handles.append(pltpu.async_copy(
                            flow_hbm.at[ib.at[gsl]], fb.at[gsl], sem))
                for h in handles:
                    h.wait()

                def grp2(g, _):
                    s = g * LANES
                    sl = pl.ds(s, LANES)
                    w00 = w00b[sl]
                    w01 = w01b[sl]
                    w10 = w10b[sl]
                    w11 = w11b[sl]
                    uo[sl] = (w00 * fu00[sl] + w01 * fu01[sl]
                              + w10 * fu10[sl] + w11 * fu11[sl])
                    vo[sl] = (w00 * fv00[sl] + w01 * fv01[sl]
                              + w10 * fv10[sl] + w11 * fv11[sl])
                    return _

                lax.fori_loop(0, n_grp, grp2, None)

                pltpu.sync_copy(uo, u_hbm.at[pl.ds(base, CH_A)])
                pltpu.sync_copy(vo, v_hbm.at[pl.ds(base, CH_A)])

            return _

        lax.fori_loop(0, n_rounds, round_body, None)

    return flow_kernel


def _hist_kernel(Bq, Nq, Dq, Hq, Wq):
    R = Dq + 1
    QROWS = Hq // 4
    PLANE = QROWS * Wq
    ACC = 2 * PLANE
    n_tasks = Bq * R * 4
    per_w = (n_tasks + NW - 1) // NW
    n_chunks = Nq // CH_B
    inv_base = 1.0 / float(Dq)
    mesh = plsc.VectorSubcoreMesh(core_axis_name="c", subcore_axis_name="s")

    @functools.partial(
        pl.kernel,
        mesh=mesh,
        out_type=jax.ShapeDtypeStruct((n_tasks * 32,), jnp.float32),
        compiler_params=pltpu.CompilerParams(needs_layout_passes=False),
        scratch_types=[pltpu.VMEM((CH_B,), jnp.float32)] * 12 + [
            pltpu.VMEM((ACC,), jnp.float32),
            pltpu.VMEM((ACC,), jnp.float32),
            pltpu.VMEM((16,), jnp.float32),
            pltpu.VMEM((16,), jnp.float32),
            pltpu.SemaphoreType.DMA,
            pltpu.SemaphoreType.DMA,
        ],
    )
    def hist_kernel(x_hbm, y_hbm, t_hbm, p_hbm, u_hbm, v_hbm, part_hbm,
                    xb0, yb0, tb0, pb0, ub0, vb0,
                    xb1, yb1, tb1, pb1, ub1, vb1,
                    eacc, tacc, sb, cb, sem0, sem1):
        wid = lax.axis_index("s") * NC + lax.axis_index("c")
        zero16 = jnp.zeros((16,), jnp.float32)
        n_grp = CH_B // LANES
        hbms = (x_hbm, y_hbm, t_hbm, p_hbm, u_hbm, v_hbm)
        bufsets = ((xb0, yb0, tb0, pb0, ub0, vb0),
                   (xb1, yb1, tb1, pb1, ub1, vb1))
        sems = (sem0, sem1)

        def fire(setn, b, ci):
            osl = pl.ds(b * Nq + ci * CH_B, CH_B)
            for h, buf in zip(hbms, bufsets[setn]):
                pltpu.async_copy(h.at[osl], buf, sems[setn])

        def drain(setn, b, ci):
            osl = pl.ds(b * Nq + ci * CH_B, CH_B)
            for h, buf in zip(hbms, bufsets[setn]):
                pltpu.make_async_copy(h.at[osl], buf, sems[setn]).wait()

        def task_body(k, _):
            task = wid + NW * k

            @pl.when(task < n_tasks)
            def _():
                b = task // (R * 4)
                rem = task % (R * 4)
                r = rem // 4
                q = rem % 4
                rf = r.astype(jnp.float32)
                ybase = q * QROWS

                fire(0, b, 0)

                def zero_body(i, _):
                    eacc[pl.ds(i * LANES, LANES)] = zero16
                    tacc[pl.ds(i * LANES, LANES)] = zero16
                    return _

                lax.fori_loop(0, ACC // LANES, zero_body, None)

                def compute(bufs):
                    xb, yb, tb, pb, ub, vb = bufs

                    def grp(g, _):
                        s = g * LANES
                        sl = pl.ds(s, LANES)
                        x = xb[sl]
                        y = yb[sl]
                        t = tb[sl]
                        p = pb[sl]
                        u = ub[sl]
                        v = vb[sl]
                        dt = rf - t
                        xw = x + dt * u
                        yw = y + dt * v
                        ts = jnp.abs(dt) * inv_base
                        inb = ((xw >= 0.0) & (xw <= float(Wq - 1))
                               & (yw >= 0.0) & (yw <= float(Hq - 1)))
                        pbase = jnp.where(p > 0.5, PLANE, 0)
                        x0 = xw.astype(jnp.int32)
                        y0 = yw.astype(jnp.int32)
                        ax = xw - x0.astype(jnp.float32)
                        ay = yw - y0.astype(jnp.float32)
                        bx = 1.0 - ax
                        by = 1.0 - ay
                        mx1 = inb & (x0 <= Wq - 2)
                        my1 = y0 <= Hq - 2
                        ly0 = y0 - ybase
                        ly1 = ly0 + 1
                        ok0 = inb & (ly0 >= 0) & (ly0 <= QROWS - 1)
                        ok1 = (ly1 >= 0) & (ly1 <= QROWS - 1)
                        m00 = ok0
                        m01 = mx1 & ok0
                        m10 = inb & my1 & ok1
                        m11 = mx1 & my1 & ok1
                        bxy = pbase + x0
                        r0 = ly0 * Wq
                        r1 = r0 + Wq
                        i00 = jnp.clip(bxy + r0, 0, ACC - 1)
                        i01 = jnp.minimum(i00 + 1, ACC - 1)
                        i10 = jnp.clip(bxy + r1, 0, ACC - 1)
                        i11 = jnp.minimum(i10 + 1, ACC - 1)
                        w00 = bx * by
                        w01 = ax * by
                        w10 = bx * ay
                        w11 = ax * ay
                        plsc.addupdate_scatter(eacc, [i00], w00, mask=m00)
                        plsc.addupdate_scatter(tacc, [i00], w00 * ts, mask=m00)
                        plsc.addupdate_scatter(eacc, [i01], w01, mask=m01)
                        plsc.addupdate_scatter(tacc, [i01], w01 * ts, mask=m01)
                        plsc.addupdate_scatter(eacc, [i10], w10, mask=m10)
                        plsc.addupdate_scatter(tacc, [i10], w10 * ts, mask=m10)
                        plsc.addupdate_scatter(eacc, [i11], w11, mask=m11)
                        plsc.addupdate_scatter(tacc, [i11], w11 * ts, mask=m11)
                        return _

                    lax.fori_loop(0, n_grp, grp, None)

                def pair_body(pair, _):
                    c0 = 2 * pair
                    drain(0, b, c0)
                    fire(1, b, c0 + 1)
                    compute(bufsets[0])
                    drain(1, b, c0 + 1)

                    @pl.when(c0 + 2 < n_chunks)
                    def _():
                        fire(0, b, c0 + 2)

                    compute(bufsets[1])
                    return _

                lax.fori_loop(0, n_chunks // 2, pair_body, None)

                def red(i, carry):
                    ssum, cnt = carry
                    sl0 = pl.ds(i * LANES, LANES)
                    sl1 = pl.ds(PLANE + i * LANES, LANES)
                    e0 = eacc[sl0]
                    e1 = eacc[sl1]
                    a0 = tacc[sl0] / (e0 + 1e-9)
                    a1 = tacc[sl1] / (e1 + 1e-9)
                    ssum = ssum + a0 * a0 + a1 * a1
                    cnt = cnt + jnp.where((e0 + e1) > 0.0, 1.0, 0.0)
                    return (ssum, cnt)

                ssum, cnt = lax.fori_loop(0, PLANE // LANES, red,
                                          (zero16, zero16))
                sb[...] = ssum
                cb[...] = cnt
                pltpu.sync_copy(sb, part_hbm.at[pl.ds(task * 32, 16)])
                pltpu.sync_copy(cb, part_hbm.at[pl.ds(task * 32 + 16, 16)])

            return _

        lax.fori_loop(0, per_w, task_body, None)

    return hist_kernel


def _combine_body(part_ref, loss_ref):
    m = part_ref[...]
    col = lax.broadcasted_iota(jnp.int32, m.shape, 1)
    is_sum = (col % 32) < 16
    s = jnp.sum(jnp.where(is_sum, m, 0.0), axis=1)
    c = jnp.sum(jnp.where(is_sum, 0.0, m), axis=1)
    loss_ref[...] = jnp.broadcast_to((s / (c + 1e-9))[:, None], m.shape)


def _combine_kernel(n_br):
    return pl.pallas_call(
        _combine_body,
        out_shape=jax.ShapeDtypeStruct((n_br, 128), jnp.float32),
    )


def kernel(events, flow_maps):
    Bq, Dq, Hq, Wq, _ = flow_maps.shape
    _, Nq, _ = events.shape
    R = Dq + 1
    BN = Bq * Nq
    xs = events[:, :, 0].reshape(BN)
    ys = events[:, :, 1].reshape(BN)
    tt = events[:, :, 2].reshape(BN)
    pp = events[:, :, 4].reshape(BN)
    fmp = (flow_maps.reshape(Bq, Dq, Hq // 128, 128, Wq, 2)
           .transpose(0, 1, 4, 2, 5, 3).reshape(-1))
    uu, vv = _flow_sample_kernel(BN, Nq, Dq, Hq, Wq)(xs, ys, tt, fmp)
    part = _hist_kernel(Bq, Nq, Dq, Hq, Wq)(xs, ys, tt, pp, uu, vv)
    loss = _combine_kernel(Bq * R)(part.reshape(Bq * R, 128))
    return loss[:, 0].reshape(Bq, R)


# speedup vs baseline: 82.5063x; 1.0526x over previous
import functools

import jax
import jax.numpy as jnp
from jax import lax
from jax.experimental import pallas as pl
from jax.experimental.pallas import tpu as pltpu
from jax.experimental.pallas import tpu_sc as plsc

NC = 2
NS = 16
NW = NC * NS
LANES = 16

CH_A = 640
GSUB = 128
CH_B = 2000


def _iota16():
    return lax.iota(jnp.int32, 16)


def _flow_sample_kernel(BN, Nq, Dq, Hq, Wq):
    n_chunks = BN // CH_A
    n_rounds = (n_chunks + NW - 1) // NW
    WSTRIDE = 512
    mesh = plsc.VectorSubcoreMesh(core_axis_name="c", subcore_axis_name="s")

    idx_t = [pltpu.VMEM((CH_A,), jnp.int32)] * 8
    fbuf_t = [pltpu.VMEM((CH_A,), jnp.float32)] * 8
    wbuf_t = [pltpu.VMEM((CH_A,), jnp.float32)] * 4

    @functools.partial(
        pl.kernel,
        mesh=mesh,
        out_type=(jax.ShapeDtypeStruct((BN,), jnp.float32),
                  jax.ShapeDtypeStruct((BN,), jnp.float32)),
        compiler_params=pltpu.CompilerParams(needs_layout_passes=False),
        scratch_types=[pltpu.VMEM((CH_A,), jnp.float32)] * 3 + idx_t + fbuf_t
        + wbuf_t + [
            pltpu.VMEM((CH_A,), jnp.float32),
            pltpu.VMEM((CH_A,), jnp.float32),
            pltpu.SemaphoreType.DMA,
        ],
    )
    def flow_kernel(x_hbm, y_hbm, t_hbm, flow_hbm, u_hbm, v_hbm,
                    xb, yb, tb,
                    iu00, iv00, iu01, iv01, iu10, iv10, iu11, iv11,
                    fu00, fv00, fu01, fv01, fu10, fv10, fu11, fv11,
                    w00b, w01b, w10b, w11b, uo, vo, sem):
        wid = lax.axis_index("s") * NC + lax.axis_index("c")
        iota = _iota16()
        n_grp = CH_A // LANES

        def round_body(j, _):
            chunk = wid + NW * j

            @pl.when(chunk < n_chunks)
            def _():
                base = chunk * CH_A
                pltpu.sync_copy(x_hbm.at[pl.ds(base, CH_A)], xb)
                pltpu.sync_copy(y_hbm.at[pl.ds(base, CH_A)], yb)
                pltpu.sync_copy(t_hbm.at[pl.ds(base, CH_A)], tb)

                def grp1(g, _):
                    s = g * LANES
                    sl = pl.ds(s, LANES)
                    x = xb[sl]
                    y = yb[sl]
                    t = tb[sl]
                    x0 = jnp.clip(x.astype(jnp.int32), 0, Wq - 2)
                    y0 = jnp.clip(y.astype(jnp.int32), 0, Hq - 2)
                    dx = jnp.clip(x - x0.astype(jnp.float32), 0.0, 1.0)
                    dy = jnp.clip(y - y0.astype(jnp.float32), 0.0, 1.0)
                    bidx = jnp.clip(t.astype(jnp.int32), 0, Dq - 1)
                    bvec = (base + iota + s) // Nq
                    col0 = (bvec * Dq + bidx) * (Wq * WSTRIDE) + x0 * WSTRIDE
                    y1 = y0 + 1
                    yt0 = ((y0 >> 7) << 8) + (y0 & 127)
                    yt1 = ((y1 >> 7) << 8) + (y1 & 127)
                    u00 = col0 + yt0
                    u10 = col0 + yt1
                    iu00[sl] = u00
                    iv00[sl] = u00 + 128
                    iu01[sl] = u00 + WSTRIDE
                    iv01[sl] = u00 + WSTRIDE + 128
                    iu10[sl] = u10
                    iv10[sl] = u10 + 128
                    iu11[sl] = u10 + WSTRIDE
                    iv11[sl] = u10 + WSTRIDE + 128
                    omdx = 1.0 - dx
                    omdy = 1.0 - dy
                    w00b[sl] = omdx * omdy
                    w01b[sl] = dx * omdy
                    w10b[sl] = omdx * dy
                    w11b[sl] = dx * dy
                    return _

                lax.fori_loop(0, n_grp, grp1, None)

                pairs = ((iu00, fu00), (iv00, fv00),
                         (iu01, fu01), (iv01, fv01),
                         (iu10, fu10), (iv10, fv10),
                         (iu11, fu11), (iv11, fv11))
                handles = []
                for ib, fb in pairs:
                    for j in range(CH_A // GSUB):
                        gsl = pl.ds(j * GSUB, GSUB)
                        handles.append(pltpu.async_copy(
                            flow_hbm.at[ib.at[gsl]], fb.at[gsl], sem))
                for h in handles:
                    h.wait()

                def grp2(g, _):
                    s = g * LANES
                    sl = pl.ds(s, LANES)
                    w00 = w00b[sl]
                    w01 = w01b[sl]
                    w10 = w10b[sl]
                    w11 = w11b[sl]
                    uo[sl] = (w00 * fu00[sl] + w01 * fu01[sl]
                              + w10 * fu10[sl] + w11 * fu11[sl])
                    vo[sl] = (w00 * fv00[sl] + w01 * fv01[sl]
                              + w10 * fv10[sl] + w11 * fv11[sl])
                    return _

                lax.fori_loop(0, n_grp, grp2, None)

                pltpu.sync_copy(uo, u_hbm.at[pl.ds(base, CH_A)])
                pltpu.sync_copy(vo, v_hbm.at[pl.ds(base, CH_A)])

            return _

        lax.fori_loop(0, n_rounds, round_body, None)

    return flow_kernel


def _hist_kernel(Bq, Nq, Dq, Hq, Wq):
    R = Dq + 1
    QROWS = Hq // 4
    PLANE = QROWS * Wq
    ACC = 2 * PLANE
    n_tasks = Bq * R * 4
    per_w = (n_tasks + NW - 1) // NW
    n_chunks = Nq // CH_B
    inv_base = 1.0 / float(Dq)
    mesh = plsc.VectorSubcoreMesh(core_axis_name="c", subcore_axis_name="s")

    @functools.partial(
        pl.kernel,
        mesh=mesh,
        out_type=jax.ShapeDtypeStruct((n_tasks * 32,), jnp.float32),
        compiler_params=pltpu.CompilerParams(needs_layout_passes=False),
        scratch_types=[pltpu.VMEM((CH_B,), jnp.float32)] * 12 + [
            pltpu.VMEM((ACC,), jnp.float32),
            pltpu.VMEM((ACC,), jnp.float32),
            pltpu.VMEM((16,), jnp.float32),
            pltpu.VMEM((16,), jnp.float32),
            pltpu.SemaphoreType.DMA,
            pltpu.SemaphoreType.DMA,
        ],
    )
    def hist_kernel(x_hbm, y_hbm, t_hbm, p_hbm, u_hbm, v_hbm, part_hbm,
                    xb0, yb0, tb0, pb0, ub0, vb0,
                    xb1, yb1, tb1, pb1, ub1, vb1,
                    eacc, tacc, sb, cb, sem0, sem1):
        wid = lax.axis_index("s") * NC + lax.axis_index("c")
        zero16 = jnp.zeros((16,), jnp.float32)
        n_grp = CH_B // LANES
        hbms = (x_hbm, y_hbm, t_hbm, p_hbm, u_hbm, v_hbm)
        bufsets = ((xb0, yb0, tb0, pb0, ub0, vb0),
                   (xb1, yb1, tb1, pb1, ub1, vb1))
        sems = (sem0, sem1)

        def fire(setn, b, ci):
            osl = pl.ds(b * Nq + ci * CH_B, CH_B)
            for h, buf in zip(hbms, bufsets[setn]):
                pltpu.async_copy(h.at[osl], buf, sems[setn])

        def drain(setn, b, ci):
            osl = pl.ds(b * Nq + ci * CH_B, CH_B)
            for h, buf in zip(hbms, bufsets[setn]):
                pltpu.make_async_copy(h.at[osl], buf, sems[setn]).wait()

        def task_body(k, _):
            task = wid + NW * k

            @pl.when(task < n_tasks)
            def _():
                b = task // (R * 4)
                rem = task % (R * 4)
                r = rem // 4
                q = rem % 4
                rf = r.astype(jnp.float32)
                ybase = q * QROWS

                fire(0, b, 0)

                def zero_body(i, _):
                    for z in range(5):
                        zsl = pl.ds((i * 5 + z) * LANES, LANES)
                        eacc[zsl] = zero16
                        tacc[zsl] = zero16
                    return _

                lax.fori_loop(0, ACC // (5 * LANES), zero_body, None)

                def compute(bufs):
                    xb, yb, tb, pb, ub, vb = bufs

                    def grp(g5, _):
                        for gz in range(5):
                            _splat(bufs, g5 * 5 + gz)
                        return _

                    def _splat(bufs, g):
                        xb, yb, tb, pb, ub, vb = bufs
                        s = g * LANES
                        sl = pl.ds(s, LANES)
                        x = xb[sl]
                        y = yb[sl]
                        t = tb[sl]
                        p = pb[sl]
                        u = ub[sl]
                        v = vb[sl]
                        dt = rf - t
                        xw = x + dt * u
                        yw = y + dt * v
                        ts = jnp.abs(dt) * inv_base
                        inb = ((xw >= 0.0) & (xw <= float(Wq - 1))
                               & (yw >= 0.0) & (yw <= float(Hq - 1)))
                        pbase = jnp.where(p > 0.5, PLANE, 0)
                        x0 = xw.astype(jnp.int32)
                        y0 = yw.astype(jnp.int32)
                        ax = xw - x0.astype(jnp.float32)
                        ay = yw - y0.astype(jnp.float32)
                        bx = 1.0 - ax
                        by = 1.0 - ay
                        mx1 = inb & (x0 <= Wq - 2)
                        my1 = y0 <= Hq - 2
                        ly0 = y0 - ybase
                        ly1 = ly0 + 1
                        ok0 = inb & (ly0 >= 0) & (ly0 <= QROWS - 1)
                        ok1 = (ly1 >= 0) & (ly1 <= QROWS - 1)
                        m00 = ok0
                        m01 = mx1 & ok0
                        m10 = inb & my1 & ok1
                        m11 = mx1 & my1 & ok1
                        bxy = pbase + x0
                        r0 = ly0 * Wq
                        r1 = r0 + Wq
                        i00 = jnp.clip(bxy + r0, 0, ACC - 1)
                        i01 = jnp.minimum(i00 + 1, ACC - 1)
                        i10 = jnp.clip(bxy + r1, 0, ACC - 1)
                        i11 = jnp.minimum(i10 + 1, ACC - 1)
                        w00 = bx * by
                        w01 = ax * by
                        w10 = bx * ay
                        w11 = ax * ay
                        plsc.addupdate_scatter(eacc, [i00], w00, mask=m00)
                        plsc.addupdate_scatter(tacc, [i00], w00 * ts, mask=m00)
                        plsc.addupdate_scatter(eacc, [i01], w01, mask=m01)
                        plsc.addupdate_scatter(tacc, [i01], w01 * ts, mask=m01)
                        plsc.addupdate_scatter(eacc, [i10], w10, mask=m10)
                        plsc.addupdate_scatter(tacc, [i10], w10 * ts, mask=m10)
                        plsc.addupdate_scatter(eacc, [i11], w11, mask=m11)
                        plsc.addupdate_scatter(tacc, [i11], w11 * ts, mask=m11)

                    lax.fori_loop(0, n_grp // 5, grp, None)

                def pair_body(pair, _):
                    c0 = 2 * pair
                    drain(0, b, c0)
                    fire(1, b, c0 + 1)
                    compute(bufsets[0])
                    drain(1, b, c0 + 1)

                    @pl.when(c0 + 2 < n_chunks)
                    def _():
                        fire(0, b, c0 + 2)

                    compute(bufsets[1])
                    return _

                lax.fori_loop(0, n_chunks // 2, pair_body, None)

                def red(i, carry):
                    ssum, cnt = carry
                    for z in range(4):
                        j = i * 4 + z
                        sl0 = pl.ds(j * LANES, LANES)
                        sl1 = pl.ds(PLANE + j * LANES, LANES)
                        e0 = eacc[sl0]
                        e1 = eacc[sl1]
                        a0 = tacc[sl0] / (e0 + 1e-9)
                        a1 = tacc[sl1] / (e1 + 1e-9)
                        ssum = ssum + a0 * a0 + a1 * a1
                        cnt = cnt + jnp.where((e0 + e1) > 0.0, 1.0, 0.0)
                    return (ssum, cnt)

                ssum, cnt = lax.fori_loop(0, PLANE // (4 * LANES), red,
                                          (zero16, zero16))
                sb[...] = ssum
                cb[...] = cnt
                pltpu.sync_copy(sb, part_hbm.at[pl.ds(task * 32, 16)])
                pltpu.sync_copy(cb, part_hbm.at[pl.ds(task * 32 + 16, 16)])

            return _

        lax.fori_loop(0, per_w, task_body, None)

    return hist_kernel


def _combine_body(part_ref, loss_ref):
    m = part_ref[...]
    col = lax.broadcasted_iota(jnp.int32, m.shape, 1)
    is_sum = (col % 32) < 16
    s = jnp.sum(jnp.where(is_sum, m, 0.0), axis=1)
    c = jnp.sum(jnp.where(is_sum, 0.0, m), axis=1)
    loss_ref[...] = jnp.broadcast_to((s / (c + 1e-9))[:, None], m.shape)


def _combine_kernel(n_br):
    return pl.pallas_call(
        _combine_body,
        out_shape=jax.ShapeDtypeStruct((n_br, 128), jnp.float32),
    )


def kernel(events, flow_maps):
    Bq, Dq, Hq, Wq, _ = flow_maps.shape
    _, Nq, _ = events.shape
    R = Dq + 1
    BN = Bq * Nq
    xs = events[:, :, 0].reshape(BN)
    ys = events[:, :, 1].reshape(BN)
    tt = events[:, :, 2].reshape(BN)
    pp = events[:, :, 4].reshape(BN)
    fmp = (flow_maps.reshape(Bq, Dq, Hq // 128, 128, Wq, 2)
           .transpose(0, 1, 4, 2, 5, 3).reshape(-1))
    uu, vv = _flow_sample_kernel(BN, Nq, Dq, Hq, Wq)(xs, ys, tt, fmp)
    part = _hist_kernel(Bq, Nq, Dq, Hq, Wq)(xs, ys, tt, pp, uu, vv)
    loss = _combine_kernel(Bq * R)(part.reshape(Bq * R, 128))
    return loss[:, 0].reshape(Bq, R)


# speedup vs baseline: 84.3730x; 1.0226x over previous
import functools

import jax
import jax.numpy as jnp
from jax import lax
from jax.experimental import pallas as pl
from jax.experimental.pallas import tpu as pltpu
from jax.experimental.pallas import tpu_sc as plsc

NC = 2
NS = 16
NW = NC * NS
LANES = 16

CH_A = 640
GSUB = 128
CH_B = 2000


def _iota16():
    return lax.iota(jnp.int32, 16)


def _flow_sample_kernel(BN, Nq, Dq, Hq, Wq):
    n_chunks = BN // CH_A
    n_rounds = (n_chunks + NW - 1) // NW
    WSTRIDE = 512
    mesh = plsc.VectorSubcoreMesh(core_axis_name="c", subcore_axis_name="s")

    nset = 25
    set_t = ([pltpu.VMEM((CH_A,), jnp.float32)] * 3
             + [pltpu.VMEM((CH_A,), jnp.int32)] * 8
             + [pltpu.VMEM((CH_A,), jnp.float32)] * 8
             + [pltpu.VMEM((CH_A,), jnp.float32)] * 4
             + [pltpu.VMEM((CH_A,), jnp.float32)] * 2)

    @functools.partial(
        pl.kernel,
        mesh=mesh,
        out_type=(jax.ShapeDtypeStruct((BN,), jnp.float32),
                  jax.ShapeDtypeStruct((BN,), jnp.float32)),
        compiler_params=pltpu.CompilerParams(needs_layout_passes=False),
        scratch_types=set_t + set_t + [
            pltpu.SemaphoreType.DMA,
            pltpu.SemaphoreType.DMA,
            pltpu.SemaphoreType.DMA,
            pltpu.SemaphoreType.DMA,
        ],
    )
    def flow_kernel(x_hbm, y_hbm, t_hbm, flow_hbm, u_hbm, v_hbm, *scr):
        sets = (scr[:nset], scr[nset:2 * nset])
        isem0, isem1, gsem0, gsem1 = scr[2 * nset:]
        isems = (isem0, isem1)
        gsems = (gsem0, gsem1)
        wid = lax.axis_index("s") * NC + lax.axis_index("c")
        iota = _iota16()
        n_grp = CH_A // LANES

        def fire_in(setn, base):
            xb, yb, tb = sets[setn][:3]
            pltpu.async_copy(x_hbm.at[pl.ds(base, CH_A)], xb, isems[setn])
            pltpu.async_copy(y_hbm.at[pl.ds(base, CH_A)], yb, isems[setn])
            pltpu.async_copy(t_hbm.at[pl.ds(base, CH_A)], tb, isems[setn])

        def drain_in(setn, base):
            xb, yb, tb = sets[setn][:3]
            for h, buf in ((x_hbm, xb), (y_hbm, yb), (t_hbm, tb)):
                pltpu.make_async_copy(h.at[pl.ds(base, CH_A)], buf,
                                      isems[setn]).wait()

        def pass1(setn, base):
            xb, yb, tb = sets[setn][:3]
            ixs = sets[setn][3:11]
            wbs = sets[setn][19:23]
            iu00, iv00, iu01, iv01, iu10, iv10, iu11, iv11 = ixs
            w00b, w01b, w10b, w11b = wbs

            def grp1(g, _):
                s = g * LANES
                sl = pl.ds(s, LANES)
                x = xb[sl]
                y = yb[sl]
                t = tb[sl]
                x0 = jnp.clip(x.astype(jnp.int32), 0, Wq - 2)
                y0 = jnp.clip(y.astype(jnp.int32), 0, Hq - 2)
                dx = jnp.clip(x - x0.astype(jnp.float32), 0.0, 1.0)
                dy = jnp.clip(y - y0.astype(jnp.float32), 0.0, 1.0)
                bidx = jnp.clip(t.astype(jnp.int32), 0, Dq - 1)
                bvec = (base + iota + s) // Nq
                col0 = (bvec * Dq + bidx) * (Wq * WSTRIDE) + x0 * WSTRIDE
                y1 = y0 + 1
                yt0 = ((y0 >> 7) << 8) + (y0 & 127)
                yt1 = ((y1 >> 7) << 8) + (y1 & 127)
                u00 = col0 + yt0
                u10 = col0 + yt1
                iu00[sl] = u00
                iv00[sl] = u00 + 128
                iu01[sl] = u00 + WSTRIDE
                iv01[sl] = u00 + WSTRIDE + 128
                iu10[sl] = u10
                iv10[sl] = u10 + 128
                iu11[sl] = u10 + WSTRIDE
                iv11[sl] = u10 + WSTRIDE + 128
                omdx = 1.0 - dx
                omdy = 1.0 - dy
                w00b[sl] = omdx * omdy
                w01b[sl] = dx * omdy
                w10b[sl] = omdx * dy
                w11b[sl] = dx * dy
                return _

            lax.fori_loop(0, n_grp, grp1, None)

        def fire_g(setn):
            ixs = sets[setn][3:11]
            fbs = sets[setn][11:19]
            for ib, fb in zip(ixs, fbs):
                for j in range(CH_A // GSUB):
                    gsl = pl.ds(j * GSUB, GSUB)
                    pltpu.async_copy(flow_hbm.at[ib.at[gsl]], fb.at[gsl],
                                     gsems[setn])

        def drain_g(setn):
            ixs = sets[setn][3:11]
            fbs = sets[setn][11:19]
            for ib, fb in zip(ixs, fbs):
                for j in range(CH_A // GSUB):
                    gsl = pl.ds(j * GSUB, GSUB)
                    pltpu.make_async_copy(flow_hbm.at[ib.at[gsl]], fb.at[gsl],
                                          gsems[setn]).wait()

        def pass2_out(setn, base):
            fbs = sets[setn][11:19]
            fu00, fv00, fu01, fv01, fu10, fv10, fu11, fv11 = fbs
            w00b, w01b, w10b, w11b = sets[setn][19:23]
            uo, vo = sets[setn][23:25]

            def grp2(g, _):
                s = g * LANES
                sl = pl.ds(s, LANES)
                w00 = w00b[sl]
                w01 = w01b[sl]
                w10 = w10b[sl]
                w11 = w11b[sl]
                uo[sl] = (w00 * fu00[sl] + w01 * fu01[sl]
                          + w10 * fu10[sl] + w11 * fu11[sl])
                vo[sl] = (w00 * fv00[sl] + w01 * fv01[sl]
                          + w10 * fv10[sl] + w11 * fv11[sl])
                return _

            lax.fori_loop(0, n_grp, grp2, None)
            pltpu.sync_copy(uo, u_hbm.at[pl.ds(base, CH_A)])
            pltpu.sync_copy(vo, v_hbm.at[pl.ds(base, CH_A)])

        fire_in(0, wid * CH_A)

        def pair_body(p, _):
            j0 = 2 * p
            j1 = j0 + 1
            c0 = wid + NW * j0
            c1 = wid + NW * j1
            b0 = c0 * CH_A
            b1 = c1 * CH_A
            v1 = c1 < n_chunks
            drain_in(0, b0)
            pass1(0, b0)
            fire_g(0)

            @pl.when(v1)
            def _():
                fire_in(1, b1)

            drain_g(0)
            pass2_out(0, b0)

            @pl.when(v1)
            def _():
                drain_in(1, b1)
                pass1(1, b1)
                fire_g(1)

            @pl.when(j0 + 2 < n_rounds)
            def _():
                fire_in(0, (wid + NW * (j0 + 2)) * CH_A)

            @pl.when(v1)
            def _():
                drain_g(1)
                pass2_out(1, b1)

            return _

        lax.fori_loop(0, n_rounds // 2, pair_body, None)

    return flow_kernel


def _hist_kernel(Bq, Nq, Dq, Hq, Wq):
    R = Dq + 1
    QROWS = Hq // 4
    PLANE = QROWS * Wq
    ACC = 2 * PLANE
    n_tasks = Bq * R * 4
    per_w = (n_tasks + NW - 1) // NW
    n_chunks = Nq // CH_B
    inv_base = 1.0 / float(Dq)
    mesh = plsc.VectorSubcoreMesh(core_axis_name="c", subcore_axis_name="s")

    @functools.partial(
        pl.kernel,
        mesh=mesh,
        out_type=jax.ShapeDtypeStruct((n_tasks * 32,), jnp.float32),
        compiler_params=pltpu.CompilerParams(needs_layout_passes=False),
        scratch_types=[pltpu.VMEM((CH_B,), jnp.float32)] * 12 + [
            pltpu.VMEM((ACC,), jnp.float32),
            pltpu.VMEM((ACC,), jnp.float32),
            pltpu.VMEM((16,), jnp.float32),
            pltpu.VMEM((16,), jnp.float32),
            pltpu.SemaphoreType.DMA,
            pltpu.SemaphoreType.DMA,
        ],
    )
    def hist_kernel(x_hbm, y_hbm, t_hbm, p_hbm, u_hbm, v_hbm, part_hbm,
                    xb0, yb0, tb0, pb0, ub0, vb0,
                    xb1, yb1, tb1, pb1, ub1, vb1,
                    eacc, tacc, sb, cb, sem0, sem1):
        wid = lax.axis_index("s") * NC + lax.axis_index("c")
        zero16 = jnp.zeros((16,), jnp.float32)
        n_grp = CH_B // LANES
        hbms = (x_hbm, y_hbm, t_hbm, p_hbm, u_hbm, v_hbm)
        bufsets = ((xb0, yb0, tb0, pb0, ub0, vb0),
                   (xb1, yb1, tb1, pb1, ub1, vb1))
        sems = (sem0, sem1)

        def fire(setn, b, ci):
            osl = pl.ds(b * Nq + ci * CH_B, CH_B)
            for h, buf in zip(hbms, bufsets[setn]):
                pltpu.async_copy(h.at[osl], buf, sems[setn])

        def drain(setn, b, ci):
            osl = pl.ds(b * Nq + ci * CH_B, CH_B)
            for h, buf in zip(hbms, bufsets[setn]):
                pltpu.make_async_copy(h.at[osl], buf, sems[setn]).wait()

        def task_body(k, _):
            task = wid + NW * k

            @pl.when(task < n_tasks)
            def _():
                b = task // (R * 4)
                rem = task % (R * 4)
                r = rem // 4
                q = rem % 4
                rf = r.astype(jnp.float32)
                ybase = q * QROWS

                fire(0, b, 0)

                def zero_body(i, _):
                    for z in range(5):
                        zsl = pl.ds((i * 5 + z) * LANES, LANES)
                        eacc[zsl] = zero16
                        tacc[zsl] = zero16
                    return _

                lax.fori_loop(0, ACC // (5 * LANES), zero_body, None)

                def compute(bufs):
                    xb, yb, tb, pb, ub, vb = bufs

                    def grp(g5, _):
                        for gz in range(5):
                            _splat(bufs, g5 * 5 + gz)
                        return _

                    def _splat(bufs, g):
                        xb, yb, tb, pb, ub, vb = bufs
                        s = g * LANES
                        sl = pl.ds(s, LANES)
                        x = xb[sl]
                        y = yb[sl]
                        t = tb[sl]
                        p = pb[sl]
                        u = ub[sl]
                        v = vb[sl]
                        dt = rf - t
                        xw = x + dt * u
                        yw = y + dt * v
                        ts = jnp.abs(dt) * inv_base
                        inb = ((xw >= 0.0) & (xw <= float(Wq - 1))
                               & (yw >= 0.0) & (yw <= float(Hq - 1)))
                        pbase = jnp.where(p > 0.5, PLANE, 0)
                        x0 = xw.astype(jnp.int32)
                        y0 = yw.astype(jnp.int32)
                        ax = xw - x0.astype(jnp.float32)
                        ay = yw - y0.astype(jnp.float32)
                        bx = 1.0 - ax
                        by = 1.0 - ay
                        mx1 = inb & (x0 <= Wq - 2)
                        my1 = y0 <= Hq - 2
                        ly0 = y0 - ybase
                        ly1 = ly0 + 1
                        ok0 = inb & (ly0 >= 0) & (ly0 <= QROWS - 1)
                        ok1 = (ly1 >= 0) & (ly1 <= QROWS - 1)
                        m00 = ok0
                        m01 = mx1 & ok0
                        m10 = inb & my1 & ok1
                        m11 = mx1 & my1 & ok1
                        bxy = pbase + x0
                        r0 = ly0 * Wq
                        r1 = r0 + Wq
                        i00 = jnp.clip(bxy + r0, 0, ACC - 1)
                        i01 = jnp.minimum(i00 + 1, ACC - 1)
                        i10 = jnp.clip(bxy + r1, 0, ACC - 1)
                        i11 = jnp.minimum(i10 + 1, ACC - 1)
                        w00 = bx * by
                        w01 = ax * by
                        w10 = bx * ay
                        w11 = ax * ay
                        plsc.addupdate_scatter(eacc, [i00], w00, mask=m00)
                        plsc.addupdate_scatter(tacc, [i00], w00 * ts, mask=m00)
                        plsc.addupdate_scatter(eacc, [i01], w01, mask=m01)
                        plsc.addupdate_scatter(tacc, [i01], w01 * ts, mask=m01)
                        plsc.addupdate_scatter(eacc, [i10], w10, mask=m10)
                        plsc.addupdate_scatter(tacc, [i10], w10 * ts, mask=m10)
                        plsc.addupdate_scatter(eacc, [i11], w11, mask=m11)
                        plsc.addupdate_scatter(tacc, [i11], w11 * ts, mask=m11)

                    lax.fori_loop(0, n_grp // 5, grp, None)

                def pair_body(pair, _):
                    c0 = 2 * pair
                    drain(0, b, c0)
                    fire(1, b, c0 + 1)
                    compute(bufsets[0])
                    drain(1, b, c0 + 1)

                    @pl.when(c0 + 2 < n_chunks)
                    def _():
                        fire(0, b, c0 + 2)

                    compute(bufsets[1])
                    return _

                lax.fori_loop(0, n_chunks // 2, pair_body, None)

                def red(i, carry):
                    ssum, cnt = carry
                    for z in range(4):
                        j = i * 4 + z
                        sl0 = pl.ds(j * LANES, LANES)
                        sl1 = pl.ds(PLANE + j * LANES, LANES)
                        e0 = eacc[sl0]
                        e1 = eacc[sl1]
                        a0 = tacc[sl0] / (e0 + 1e-9)
                        a1 = tacc[sl1] / (e1 + 1e-9)
                        ssum = ssum + a0 * a0 + a1 * a1
                        cnt = cnt + jnp.where((e0 + e1) > 0.0, 1.0, 0.0)
                    return (ssum, cnt)

                ssum, cnt = lax.fori_loop(0, PLANE // (4 * LANES), red,
                                          (zero16, zero16))
                sb[...] = ssum
                cb[...] = cnt
                pltpu.sync_copy(sb, part_hbm.at[pl.ds(task * 32, 16)])
                pltpu.sync_copy(cb, part_hbm.at[pl.ds(task * 32 + 16, 16)])

            return _

        lax.fori_loop(0, per_w, task_body, None)

    return hist_kernel


def _combine_body(part_ref, loss_ref):
    m = part_ref[...]
    col = lax.broadcasted_iota(jnp.int32, m.shape, 1)
    is_sum = (col % 32) < 16
    s = jnp.sum(jnp.where(is_sum, m, 0.0), axis=1)
    c = jnp.sum(jnp.where(is_sum, 0.0, m), axis=1)
    loss_ref[...] = jnp.broadcast_to((s / (c + 1e-9))[:, None], m.shape)


def _combine_kernel(n_br):
    return pl.pallas_call(
        _combine_body,
        out_shape=jax.ShapeDtypeStruct((n_br, 128), jnp.float32),
    )


def kernel(events, flow_maps):
    Bq, Dq, Hq, Wq, _ = flow_maps.shape
    _, Nq, _ = events.shape
    R = Dq + 1
    BN = Bq * Nq
    xs = events[:, :, 0].reshape(BN)
    ys = events[:, :, 1].reshape(BN)
    tt = events[:, :, 2].reshape(BN)
    pp = events[:, :, 4].reshape(BN)
    fmp = (flow_maps.reshape(Bq, Dq, Hq // 128, 128, Wq, 2)
           .transpose(0, 1, 4, 2, 5, 3).reshape(-1))
    uu, vv = _flow_sample_kernel(BN, Nq, Dq, Hq, Wq)(xs, ys, tt, fmp)
    part = _hist_kernel(Bq, Nq, Dq, Hq, Wq)(xs, ys, tt, pp, uu, vv)
    loss = _combine_kernel(Bq * R)(part.reshape(Bq * R, 128))
    return loss[:, 0].reshape(Bq, R)
